# Initial kernel scaffold; baseline (speedup 1.0000x reference)
import jax
import jax.numpy as jnp
from jax.experimental import pallas as pl


def kernel(sparse_inputs, dense_inputs, emb1, emb2, W_ld, b_ld, W_a1, b_a1, W_a2, b_a2, W_f, b_f, W_l, b_l):
    raise NotImplementedError("write your pallas kernel here")


# speedup vs baseline: 1.7906x; 1.7906x over previous
import functools

import jax
import jax.numpy as jnp
from jax import lax
from jax.experimental import pallas as pl
from jax.experimental.pallas import tpu as pltpu
from jax.experimental.pallas import tpu_sc as plsc

_NC = 2
_NS = 16
_NW = _NC * _NS
_CH = 128


def _sc_gather(flat_idx, row16, emb2_flat, emb1_rows):
  FB = flat_idx.shape[0]
  E = emb2_flat.shape[1]
  rows_per_w = FB // _NW
  n_chunks = rows_per_w // _CH
  assert rows_per_w % _CH == 0

  mesh = plsc.VectorSubcoreMesh(core_axis_name="c", subcore_axis_name="s")

  @functools.partial(
      pl.kernel,
      out_type=(
          jax.ShapeDtypeStruct((FB, E), jnp.float32),
          jax.ShapeDtypeStruct((FB, 16), jnp.float32),
      ),
      mesh=mesh,
      compiler_params=pltpu.CompilerParams(use_tc_tiling_on_sc=False),
      scratch_types=[
          pltpu.VMEM((_CH,), jnp.int32),
          pltpu.VMEM((_CH,), jnp.int32),
          pltpu.VMEM((_CH, E), jnp.float32),
          pltpu.VMEM((_CH, 16), jnp.float32),
          pltpu.SemaphoreType.DMA,
          pltpu.SemaphoreType.DMA,
      ],
  )
  def sc_k(fidx_hbm, ridx_hbm, emb2_hbm, e1_hbm, g_out, v_out,
           fidx_v, ridx_v, rows_v, vals_v, sem_a, sem_b):
    wid = lax.axis_index("s") * _NC + lax.axis_index("c")
    base = wid * rows_per_w

    def body(c, carry):
      off = base + c * _CH
      pltpu.sync_copy(fidx_hbm.at[pl.ds(off, _CH)], fidx_v)
      pltpu.sync_copy(ridx_hbm.at[pl.ds(off, _CH)], ridx_v)
      cp_a = pltpu.async_copy(emb2_hbm.at[fidx_v], rows_v, sem_a)
      cp_b = pltpu.async_copy(e1_hbm.at[ridx_v], vals_v, sem_b)
      cp_a.wait()
      cp_b.wait()
      pltpu.sync_copy(rows_v, g_out.at[pl.ds(off, _CH)])
      pltpu.sync_copy(vals_v, v_out.at[pl.ds(off, _CH)])
      return carry

    lax.fori_loop(0, n_chunks, body, 0)

  return sc_k(flat_idx, row16, emb2_flat, emb1_rows)


def _tc_body(g_ref, v16_ref, lo_ref, d_ref, wld_ref, bld_ref, wa1_ref,
             ba1_ref, wa2_ref, ba2_ref, wf_ref, bf_ref, wl_ref, bl_ref,
             of_ref, ol_ref):
  Fdim, BB, E = g_ref.shape
  gv = g_ref[...]
  wa1 = wa1_ref[...]
  ba1 = ba1_ref[...]
  wa2 = wa2_ref[...].reshape(1, 1, E)

  chunks = [[1]] + [[d, Fdim + 1 - d] for d in range(2, Fdim // 2 + 1)]
  pc = Fdim - 1

  m = jnp.full((BB,), -jnp.inf, dtype=jnp.float32)
  den = jnp.zeros((BB,), dtype=jnp.float32)
  num = jnp.zeros((BB,), dtype=jnp.float32)
  for ds in chunks:
    prods = [(gv[:Fdim - d] * gv[d:]).reshape((Fdim - d) * BB, E)
             for d in ds]
    p2 = prods[0] if len(prods) == 1 else jnp.concatenate(prods, axis=0)
    z = jnp.maximum(
        jnp.dot(p2, wa1, preferred_element_type=jnp.float32) + ba1, 0.0)
    z3 = z.reshape(pc, BB, E)
    p3 = p2.reshape(pc, BB, E)
    s_d = jnp.sum(z3 * wa2, axis=2)
    rs_d = jnp.sum(p3, axis=2)
    m_new = jnp.maximum(m, jnp.max(s_d, axis=0))
    scale = jnp.exp(m - m_new)
    e_d = jnp.exp(s_d - m_new[None, :])
    den = den * scale + jnp.sum(e_d, axis=0)
    num = num * scale + jnp.sum(e_d * rs_d, axis=0)
    m = m_new

  afm = num / den
  lo = lo_ref[...]
  sel = lax.broadcasted_iota(jnp.int32, (Fdim, BB, 16), 2) == lo[:, :, None]
  first = jnp.sum(jnp.sum(jnp.where(sel, v16_ref[...], 0.0), axis=2), axis=0)
  ld = jnp.sum(d_ref[...] * wld_ref[...], axis=0) + bld_ref[0]
  logits = ld + first + afm
  of_ref[...] = jax.nn.sigmoid(logits * wf_ref[0] + bf_ref[0])
  ol_ref[...] = jax.nn.sigmoid(logits * wl_ref[0] + bl_ref[0])


def _tc_afm(g, v16, lo16, dense_T, W_ld, b_ld, W_a1, b_a1, W_a2, b_a2,
            W_f, b_f, W_l, b_l, interpret=False):
  Fdim, B, E = g.shape
  D = dense_T.shape[0]
  BB = 256
  grid = B // BB
  smem = pl.BlockSpec(memory_space=pltpu.SMEM)
  return pl.pallas_call(
      _tc_body,
      grid=(grid,),
      in_specs=[
          pl.BlockSpec((Fdim, BB, E), lambda i: (0, i, 0)),
          pl.BlockSpec((Fdim, BB, 16), lambda i: (0, i, 0)),
          pl.BlockSpec((Fdim, BB), lambda i: (0, i)),
          pl.BlockSpec((D, BB), lambda i: (0, i)),
          pl.BlockSpec((D, 1), lambda i: (0, 0)),
          smem,
          pl.BlockSpec((E, E), lambda i: (0, 0)),
          pl.BlockSpec((1, E), lambda i: (0, 0)),
          pl.BlockSpec((1, E), lambda i: (0, 0)),
          smem,
          smem,
          smem,
          smem,
          smem,
      ],
      out_specs=[
          pl.BlockSpec((BB,), lambda i: (i,)),
          pl.BlockSpec((BB,), lambda i: (i,)),
      ],
      out_shape=[jax.ShapeDtypeStruct((B,), jnp.float32)] * 2,
      interpret=interpret,
  )(g, v16, lo16, dense_T, W_ld, b_ld, W_a1, b_a1.reshape(1, E),
    W_a2.reshape(1, E), b_a2, W_f.reshape(1), b_f, W_l.reshape(1), b_l)


def kernel(sparse_inputs, dense_inputs, emb1, emb2, W_ld, b_ld, W_a1, b_a1,
           W_a2, b_a2, W_f, b_f, W_l, b_l):
  F, V, E = emb2.shape
  B = sparse_inputs.shape[0]

  idxT = sparse_inputs.T.astype(jnp.int32)
  flat_idx = (idxT + (jnp.arange(F, dtype=jnp.int32) * V)[:, None]).reshape(-1)
  row16 = flat_idx // 16
  lo16 = (flat_idx % 16).reshape(F, B)
  emb2_flat = emb2.reshape(F * V, E)
  emb1_rows = emb1.reshape(F * V // 16, 16)

  g_flat, v_flat = _sc_gather(flat_idx, row16, emb2_flat, emb1_rows)
  g = g_flat.reshape(F, B, E)
  v16 = v_flat.reshape(F, B, 16)

  fo, lo = _tc_afm(g, v16, lo16, dense_inputs.T, W_ld, b_ld, W_a1, b_a1,
                   W_a2, b_a2, W_f, b_f, W_l, b_l)
  return fo.reshape(B, 1), lo.reshape(B, 1)
